# baseline (speedup 1.0000x reference)
import functools

import numpy as np
import jax
import jax.numpy as jnp
from jax import lax
from jax.experimental import pallas as pl
from jax.experimental.pallas import tpu as pltpu
from jax.experimental.pallas import tpu_sc as plsc

A = 64
V = 32
L = 2048
B = 4096

NW = 32
RPT = B // NW
CH = 4
NCH = RPT // CH


def _build_perms():
    pk = jax.random.key(7)
    keys = jax.random.split(pk, B)
    return jax.vmap(lambda k: jax.random.permutation(k, A))(keys)


_PERMS_NP = None


def _perms_flat_operand():
    global _PERMS_NP
    if _PERMS_NP is None:
        try:
            with jax.default_device(jax.devices("cpu")[0]):
                p = jax.jit(_build_perms)()
                _PERMS_NP = np.asarray(jax.device_get(p)).astype(np.int32).reshape(-1)
        except Exception:
            return _build_perms().reshape(-1).astype(jnp.int32)
    return jnp.asarray(_PERMS_NP)


def _row_fill(xin, perms_v, syms_v, vals_row, obuf, lr, r):
    xrb = r * L
    rb = r * L
    lane = jnp.arange(16, dtype=jnp.int32)
    for k in range(A // 16):
        gbase = lane * V + (xrb + 16 * V * k)
        m = plsc.load_gather(xin, [gbase])
        mi = jnp.zeros((16,), jnp.int32)
        for j in range(1, V):
            c = plsc.load_gather(xin, [gbase + j])
            take = c > m
            m = jnp.where(take, c, m)
            mi = jnp.where(take, jnp.int32(j), mi)
        vals_row[pl.ds(16 * k, 16)] = mi
    carry = jnp.int32(0)
    for k in range(A // 16):
        pv = perms_v[pl.ds(lr * A + 16 * k, 16)]
        lens = plsc.load_gather(vals_row, [pv])
        syms = plsc.load_gather(syms_v, [pv])
        ends = plsc.cumsum(lens) + carry
        carry = carry + jnp.sum(lens)
        starts = ends - lens
        for o in range(V - 1):
            plsc.store_scatter(obuf, [starts + (rb + o)], syms,
                               mask=lens > o)


def _fill_body(x_hbm, perms_hbm, syms_hbm, out_hbm,
               xin0, xin1, obuf0, obuf1, perms_v, syms_v, vals_row,
               isem0, isem1, osem0, osem1):
    wid = lax.axis_index("c") * 16 + lax.axis_index("s")
    base = wid * RPT
    xins = (xin0, xin1)
    obufs = (obuf0, obuf1)
    isems = (isem0, isem1)
    osems = (osem0, osem1)

    pltpu.sync_copy(perms_hbm.at[pl.ds(base * A, RPT * A)], perms_v)
    pltpu.sync_copy(syms_hbm, syms_v)

    def in_src(k):
        return x_hbm.at[pl.ds((base + k * CH) * L, CH * L)]

    def out_dst(k):
        return out_hbm.at[pl.ds((base + k * CH) * L, CH * L)]

    pltpu.async_copy(in_src(0), xin0, isem0)

    zero16 = jnp.zeros((16,), jnp.int32)

    @pl.loop(0, NCH // 2)
    def _pair(c):
        for b in range(2):
            k = c * 2 + b
            @pl.when(k < NCH - 1)
            def _():
                pltpu.async_copy(in_src(k + 1), xins[1 - b], isems[1 - b])
            @pl.when(k >= 2)
            def _():
                pltpu.make_async_copy(obufs[b], out_dst(k), osems[b]).wait()
            pltpu.make_async_copy(in_src(k), xins[b], isems[b]).wait()
            for i in range(CH * L // 16):
                obufs[b][pl.ds(i * 16, 16)] = zero16
            for r in range(CH):
                lr = k * CH + r
                _row_fill(xins[b], perms_v, syms_v, vals_row, obufs[b], lr, r)
            pltpu.async_copy(obufs[b], out_dst(k), osems[b])

    pltpu.make_async_copy(obuf0, out_dst(NCH - 2), osem0).wait()
    pltpu.make_async_copy(obuf1, out_dst(NCH - 1), osem1).wait()


def _sc_bos(x_flat, perms_flat, symbols):
    kern = functools.partial(
        pl.kernel,
        out_type=jax.ShapeDtypeStruct((B * L,), jnp.int32),
        mesh=plsc.VectorSubcoreMesh(core_axis_name="c", subcore_axis_name="s"),
        scratch_types=[
            pltpu.VMEM((CH * L,), jnp.float32),
            pltpu.VMEM((CH * L,), jnp.float32),
            pltpu.VMEM((CH * L,), jnp.int32),
            pltpu.VMEM((CH * L,), jnp.int32),
            pltpu.VMEM((RPT * A,), jnp.int32),
            pltpu.VMEM((A,), jnp.int32),
            pltpu.VMEM((A,), jnp.int32),
            pltpu.SemaphoreType.DMA,
            pltpu.SemaphoreType.DMA,
            pltpu.SemaphoreType.DMA,
            pltpu.SemaphoreType.DMA,
        ],
        compiler_params=pltpu.CompilerParams(needs_layout_passes=False),
    )(_fill_body)
    return kern(x_flat, perms_flat, symbols)


def kernel(x, symbols):
    perms_flat = _perms_flat_operand()
    out_flat = _sc_bos(x.reshape(-1), perms_flat, symbols)
    result = out_flat.reshape(B, L)
    zeros = jnp.zeros((B, L), jnp.float32)
    return (result, zeros, zeros)


# speedup vs baseline: 1.7768x; 1.7768x over previous
import functools

import numpy as np
import jax
import jax.numpy as jnp
from jax import lax
from jax.experimental import pallas as pl
from jax.experimental.pallas import tpu as pltpu
from jax.experimental.pallas import tpu_sc as plsc

A = 64
V = 32
L = 2048
B = 4096

NW = 32
RPT = B // NW
CH = 8
NCH = RPT // CH


def _build_perms():
    pk = jax.random.key(7)
    keys = jax.random.split(pk, B)
    return jax.vmap(lambda k: jax.random.permutation(k, A))(keys)


_PERMS_NP = None


def _perms_flat_operand():
    global _PERMS_NP
    if _PERMS_NP is None:
        try:
            with jax.default_device(jax.devices("cpu")[0]):
                p = jax.jit(_build_perms)()
                _PERMS_NP = np.asarray(jax.device_get(p)).astype(np.int32).reshape(-1)
        except Exception:
            return _build_perms().reshape(-1).astype(jnp.int32)
    return jnp.asarray(_PERMS_NP)


def _row_fill(xin, perms_v, syms_v, vals_row, obuf, lr, r):
    xrb = r * L
    rb = r * L
    zero16 = jnp.zeros((16,), jnp.int32)
    for i in range(L // 16):
        obuf[pl.ds(rb + i * 16, 16)] = zero16
    lane = lax.iota(jnp.int32, 16)
    for k in range(A // 16):
        gbase = lane * V + (xrb + 16 * V * k)
        m = plsc.load_gather(xin, [gbase + lane])
        mi = lane
        for t in range(1, V):
            jv = (lane + t) & (V - 1)
            c = plsc.load_gather(xin, [gbase + jv])
            take = (c > m) | ((c == m) & (jv < mi))
            m = jnp.where(take, c, m)
            mi = jnp.where(take, jv, mi)
        vals_row[pl.ds(16 * k, 16)] = mi
    carry = jnp.int32(0)
    for k in range(A // 16):
        pv = perms_v[pl.ds(lr * A + 16 * k, 16)]
        lens = plsc.load_gather(vals_row, [pv])
        syms = plsc.load_gather(syms_v, [pv])
        ends = plsc.cumsum(lens) + carry
        carry = carry + jnp.sum(lens)
        starts = ends - lens
        for o in range(V - 1):
            plsc.store_scatter(obuf, [starts + (rb + o)], syms,
                               mask=lens > o)


def _fill_body(x_hbm, perms_hbm, syms_hbm, out_hbm,
               xin0, xin1, obuf0, obuf1, perms_v, syms_v, vals_row,
               isem0, isem1, osem0, osem1):
    wid = lax.axis_index("c") * 16 + lax.axis_index("s")
    base = wid * RPT
    xins = (xin0, xin1)
    obufs = (obuf0, obuf1)
    isems = (isem0, isem1)
    osems = (osem0, osem1)

    pltpu.sync_copy(perms_hbm.at[pl.ds(base * A, RPT * A)], perms_v)
    pltpu.sync_copy(syms_hbm, syms_v)

    def in_src(k):
        return x_hbm.at[pl.ds((base + k * CH) * L, CH * L)]

    def out_dst(k):
        return out_hbm.at[pl.ds((base + k * CH) * L, CH * L)]

    pltpu.async_copy(in_src(0), xin0, isem0)

    @pl.loop(0, NCH // 2)
    def _pair(c):
        for b in range(2):
            k = c * 2 + b
            @pl.when(k < NCH - 1)
            def _():
                pltpu.async_copy(in_src(k + 1), xins[1 - b], isems[1 - b])
            @pl.when(k >= 2)
            def _():
                pltpu.make_async_copy(obufs[b], out_dst(k), osems[b]).wait()
            pltpu.make_async_copy(in_src(k), xins[b], isems[b]).wait()
            xin_b, obuf_b = xins[b], obufs[b]

            @pl.loop(0, CH)
            def _row(r):
                lr = k * CH + r
                _row_fill(xin_b, perms_v, syms_v, vals_row, obuf_b, lr, r)

            pltpu.async_copy(obufs[b], out_dst(k), osems[b])

    pltpu.make_async_copy(obuf0, out_dst(NCH - 2), osem0).wait()
    pltpu.make_async_copy(obuf1, out_dst(NCH - 1), osem1).wait()


def _sc_bos(x_flat, perms_flat, symbols):
    kern = functools.partial(
        pl.kernel,
        out_type=jax.ShapeDtypeStruct((B * L,), jnp.int32),
        mesh=plsc.VectorSubcoreMesh(core_axis_name="c", subcore_axis_name="s"),
        scratch_types=[
            pltpu.VMEM((CH * L,), jnp.float32),
            pltpu.VMEM((CH * L,), jnp.float32),
            pltpu.VMEM((CH * L,), jnp.int32),
            pltpu.VMEM((CH * L,), jnp.int32),
            pltpu.VMEM((RPT * A,), jnp.int32),
            pltpu.VMEM((A,), jnp.int32),
            pltpu.VMEM((A,), jnp.int32),
            pltpu.SemaphoreType.DMA,
            pltpu.SemaphoreType.DMA,
            pltpu.SemaphoreType.DMA,
            pltpu.SemaphoreType.DMA,
        ],
        compiler_params=pltpu.CompilerParams(needs_layout_passes=False),
    )(_fill_body)
    return kern(x_flat, perms_flat, symbols)


def kernel(x, symbols):
    perms_flat = _perms_flat_operand()
    out_flat = _sc_bos(x.reshape(-1), perms_flat, symbols)
    result = out_flat.reshape(B, L)
    zeros = jnp.zeros((B, L), jnp.float32)
    return (result, zeros, zeros)


# speedup vs baseline: 2.0453x; 1.1511x over previous
import functools

import numpy as np
import jax
import jax.numpy as jnp
from jax import lax
from jax.experimental import pallas as pl
from jax.experimental.pallas import tpu as pltpu
from jax.experimental.pallas import tpu_sc as plsc

A = 64
V = 32
L = 2048
B = 4096

NW = 32
RPT = B // NW
CH = 8
NCH = RPT // CH


def _build_perms():
    pk = jax.random.key(7)
    keys = jax.random.split(pk, B)
    return jax.vmap(lambda k: jax.random.permutation(k, A))(keys)


_PERMS_NP = None


def _perms_flat_operand():
    global _PERMS_NP
    if _PERMS_NP is None:
        try:
            with jax.default_device(jax.devices("cpu")[0]):
                p = jax.jit(_build_perms)()
                _PERMS_NP = np.asarray(jax.device_get(p)).astype(np.int32).reshape(-1)
        except Exception:
            return _build_perms().reshape(-1).astype(jnp.int32)
    return jnp.asarray(_PERMS_NP)


def _row_fill(xin, perms_v, syms_v, vals_row, obuf, lr, r):
    zero16 = jnp.zeros((16,), jnp.int32)
    for i in range(L // 16):
        obuf[r, pl.ds(i * 16, 16)] = zero16
    lane = lax.iota(jnp.int32, 16)
    rvec = lane * 0 + r
    for k in range(A // 16):
        gbase = lane * V + 16 * V * k
        m = plsc.load_gather(xin, [rvec, gbase + lane])
        mi = lane
        for t in range(1, V):
            jv = (lane + t) & (V - 1)
            c = plsc.load_gather(xin, [rvec, gbase + jv])
            take = (c > m) | ((c == m) & (jv < mi))
            m = jnp.where(take, c, m)
            mi = jnp.where(take, jv, mi)
        vals_row[pl.ds(16 * k, 16)] = mi
    carry = jnp.int32(0)
    for k in range(A // 16):
        pv = perms_v[pl.ds(lr * A + 16 * k, 16)]
        lens = plsc.load_gather(vals_row, [pv])
        syms = plsc.load_gather(syms_v, [pv])
        ends = plsc.cumsum(lens) + carry
        carry = carry + jnp.sum(lens)
        starts = ends - lens
        for o in range(V - 1):
            plsc.store_scatter(obuf, [rvec, starts + o], syms,
                               mask=lens > o)


def _fill_body(x_hbm, perms_hbm, syms_hbm, out_hbm,
               xin0, xin1, obuf0, obuf1, perms_v, syms_v, vals_row,
               isem0, isem1, osem0, osem1):
    wid = lax.axis_index("c") * 16 + lax.axis_index("s")
    base = wid * RPT
    xins = (xin0, xin1)
    obufs = (obuf0, obuf1)
    isems = (isem0, isem1)
    osems = (osem0, osem1)

    pltpu.sync_copy(perms_hbm.at[pl.ds(base * A, RPT * A)], perms_v)
    pltpu.sync_copy(syms_hbm, syms_v)

    def in_src(k):
        return x_hbm.at[pl.ds(base + k * CH, CH), :]

    def out_dst(k):
        return out_hbm.at[pl.ds(base + k * CH, CH), :]

    pltpu.async_copy(in_src(0), xin0, isem0)

    @pl.loop(0, NCH // 2)
    def _pair(c):
        for b in range(2):
            k = c * 2 + b
            @pl.when(k < NCH - 1)
            def _():
                pltpu.async_copy(in_src(k + 1), xins[1 - b], isems[1 - b])
            @pl.when(k >= 2)
            def _():
                pltpu.make_async_copy(obufs[b], out_dst(k), osems[b]).wait()
            pltpu.make_async_copy(in_src(k), xins[b], isems[b]).wait()
            xin_b, obuf_b = xins[b], obufs[b]

            @pl.loop(0, CH)
            def _row(r):
                lr = k * CH + r
                _row_fill(xin_b, perms_v, syms_v, vals_row, obuf_b, lr, r)

            pltpu.async_copy(obufs[b], out_dst(k), osems[b])

    pltpu.make_async_copy(obuf0, out_dst(NCH - 2), osem0).wait()
    pltpu.make_async_copy(obuf1, out_dst(NCH - 1), osem1).wait()


def _sc_bos(x, perms_flat, symbols):
    kern = functools.partial(
        pl.kernel,
        out_type=jax.ShapeDtypeStruct((B, L), jnp.int32),
        mesh=plsc.VectorSubcoreMesh(core_axis_name="c", subcore_axis_name="s"),
        scratch_types=[
            pltpu.VMEM((CH, L), jnp.float32),
            pltpu.VMEM((CH, L), jnp.float32),
            pltpu.VMEM((CH, L), jnp.int32),
            pltpu.VMEM((CH, L), jnp.int32),
            pltpu.VMEM((RPT * A,), jnp.int32),
            pltpu.VMEM((A,), jnp.int32),
            pltpu.VMEM((A,), jnp.int32),
            pltpu.SemaphoreType.DMA,
            pltpu.SemaphoreType.DMA,
            pltpu.SemaphoreType.DMA,
            pltpu.SemaphoreType.DMA,
        ],
        compiler_params=pltpu.CompilerParams(needs_layout_passes=False),
    )(_fill_body)
    return kern(x, perms_flat, symbols)


def kernel(x, symbols):
    perms_flat = _perms_flat_operand()
    result = _sc_bos(x, perms_flat, symbols)
    zeros = jnp.zeros((B, L), jnp.float32)
    return (result, zeros, zeros)


# speedup vs baseline: 2.3942x; 1.1706x over previous
import functools

import numpy as np
import jax
import jax.numpy as jnp
from jax import lax
from jax.experimental import pallas as pl
from jax.experimental.pallas import tpu as pltpu
from jax.experimental.pallas import tpu_sc as plsc

A = 64
V = 32
L = 2048
B = 4096

NW = 32
RPT = B // NW
CH = 8
NCH = RPT // CH


def _build_perms():
    pk = jax.random.key(7)
    keys = jax.random.split(pk, B)
    return jax.vmap(lambda k: jax.random.permutation(k, A))(keys)


_PERMS_NP = None


def _perms_flat_operand():
    global _PERMS_NP
    if _PERMS_NP is None:
        try:
            with jax.default_device(jax.devices("cpu")[0]):
                p = jax.jit(_build_perms)()
                _PERMS_NP = np.asarray(jax.device_get(p)).astype(np.int32).reshape(-1)
        except Exception:
            return _build_perms().reshape(-1).astype(jnp.int32)
    return jnp.asarray(_PERMS_NP)


def _row_fill(xin, perms_v, syms_v, vals_row, obuf, lr, r):
    xrb = r * L
    rb = r * L
    zero16 = jnp.zeros((16,), jnp.int32)
    for i in range(L // 16):
        obuf[pl.ds(rb + i * 16, 16)] = zero16
    lane = lax.iota(jnp.int32, 16)
    for k in range(A // 16):
        gbase = lane * V + (xrb + 16 * V * k)
        m = plsc.load_gather(xin, [gbase + lane])
        mi = lane
        for t in range(1, V):
            jv = (lane + t) & (V - 1)
            c = plsc.load_gather(xin, [gbase + jv])
            take = (c > m) | ((c == m) & (jv < mi))
            m = jnp.where(take, c, m)
            mi = jnp.where(take, jv, mi)
        vals_row[pl.ds(16 * k, 16)] = mi
    carry = jnp.int32(0)
    for k in range(A // 16):
        pv = perms_v[pl.ds(lr * A + 16 * k, 16)]
        lens = plsc.load_gather(vals_row, [pv])
        syms = plsc.load_gather(syms_v, [pv])
        ends = plsc.cumsum(lens) + carry
        carry = carry + jnp.sum(lens)
        starts = ends - lens
        for o in range(V - 1):
            plsc.store_scatter(obuf, [starts + (rb + o)], syms,
                               mask=lens > o)


def _fill_body(x_hbm, perms_hbm, syms_hbm, out_hbm,
               xin0, xin1, obuf0, obuf1, perms_v, syms_v, vals_row,
               isem0, isem1, osem0, osem1):
    wid = lax.axis_index("c") * 16 + lax.axis_index("s")
    base = wid * RPT
    xins = (xin0, xin1)
    obufs = (obuf0, obuf1)
    isems = (isem0, isem1)
    osems = (osem0, osem1)

    pltpu.sync_copy(perms_hbm.at[pl.ds(base * A, RPT * A)], perms_v)
    pltpu.sync_copy(syms_hbm, syms_v)

    def start_in(k, xin, isem):
        for i in range(CH):
            pltpu.async_copy(x_hbm.at[base + k * CH + i],
                             xin.at[pl.ds(i * L, L)], isem)

    def wait_in(k, xin, isem):
        for i in range(CH):
            pltpu.make_async_copy(x_hbm.at[base + k * CH + i],
                                  xin.at[pl.ds(i * L, L)], isem).wait()

    def start_out(k, obuf, osem):
        for i in range(CH):
            pltpu.async_copy(obuf.at[pl.ds(i * L, L)],
                             out_hbm.at[base + k * CH + i], osem)

    def wait_out(k, obuf, osem):
        for i in range(CH):
            pltpu.make_async_copy(obuf.at[pl.ds(i * L, L)],
                                  out_hbm.at[base + k * CH + i], osem).wait()

    start_in(0, xin0, isem0)

    @pl.loop(0, NCH // 2)
    def _pair(c):
        for b in range(2):
            k = c * 2 + b
            @pl.when(k < NCH - 1)
            def _():
                start_in(k + 1, xins[1 - b], isems[1 - b])
            @pl.when(k >= 2)
            def _():
                wait_out(k, obufs[b], osems[b])
            wait_in(k, xins[b], isems[b])
            xin_b, obuf_b = xins[b], obufs[b]

            @pl.loop(0, CH)
            def _row(r):
                lr = k * CH + r
                _row_fill(xin_b, perms_v, syms_v, vals_row, obuf_b, lr, r)

            start_out(k, obufs[b], osems[b])

    wait_out(NCH - 2, obuf0, osem0)
    wait_out(NCH - 1, obuf1, osem1)


def _sc_bos(x, perms_flat, symbols):
    kern = functools.partial(
        pl.kernel,
        out_type=jax.ShapeDtypeStruct((B, L), jnp.int32),
        mesh=plsc.VectorSubcoreMesh(core_axis_name="c", subcore_axis_name="s"),
        scratch_types=[
            pltpu.VMEM((CH * L,), jnp.float32),
            pltpu.VMEM((CH * L,), jnp.float32),
            pltpu.VMEM((CH * L,), jnp.int32),
            pltpu.VMEM((CH * L,), jnp.int32),
            pltpu.VMEM((RPT * A,), jnp.int32),
            pltpu.VMEM((A,), jnp.int32),
            pltpu.VMEM((A,), jnp.int32),
            pltpu.SemaphoreType.DMA,
            pltpu.SemaphoreType.DMA,
            pltpu.SemaphoreType.DMA,
            pltpu.SemaphoreType.DMA,
        ],
        compiler_params=pltpu.CompilerParams(needs_layout_passes=False),
    )(_fill_body)
    return kern(x, perms_flat, symbols)


def kernel(x, symbols):
    perms_flat = _perms_flat_operand()
    result = _sc_bos(x, perms_flat, symbols)
    zeros = jnp.zeros((B, L), jnp.float32)
    return (result, zeros, zeros)


# speedup vs baseline: 2.8197x; 1.1777x over previous
import functools

import numpy as np
import jax
import jax.numpy as jnp
from jax import lax
from jax.experimental import pallas as pl
from jax.experimental.pallas import tpu as pltpu
from jax.experimental.pallas import tpu_sc as plsc

A = 64
V = 32
L = 2048
B = 4096

NW = 32
RPT = B // NW
CH = 8
NCH = RPT // CH


def _build_perms():
    pk = jax.random.key(7)
    keys = jax.random.split(pk, B)
    return jax.vmap(lambda k: jax.random.permutation(k, A))(keys)


_PERMS_NP = None


def _perms_flat_operand():
    global _PERMS_NP
    if _PERMS_NP is None:
        try:
            with jax.default_device(jax.devices("cpu")[0]):
                p = jax.jit(_build_perms)()
                _PERMS_NP = np.asarray(jax.device_get(p)).astype(np.int32).reshape(-1)
        except Exception:
            return _build_perms().reshape(-1).astype(jnp.int32)
    return jnp.asarray(_PERMS_NP)


def _row_fill(xin, perms_v, syms_v, vals_row, obuf, lr, r):
    xrb = r * L
    rb = r * L
    zero16 = jnp.zeros((16,), jnp.int32)
    for i in range(L // 16):
        obuf[pl.ds(rb + i * 16, 16)] = zero16
    lane = lax.iota(jnp.int32, 16)
    for k in range(A // 16):
        gbase = lane * V + (xrb + 16 * V * k)
        m = plsc.load_gather(xin, [gbase + lane])
        mi = lane
        for t in range(1, V):
            jv = (lane + t) & (V - 1)
            c = plsc.load_gather(xin, [gbase + jv])
            take = (c > m) | ((c == m) & (jv < mi))
            m = jnp.where(take, c, m)
            mi = jnp.where(take, jv, mi)
        vals_row[pl.ds(16 * k, 16)] = mi
    carry = jnp.int32(0)
    for k in range(A // 16):
        pv = perms_v[pl.ds(lr * A + 16 * k, 16)]
        lens = plsc.load_gather(vals_row, [pv])
        syms = plsc.load_gather(syms_v, [pv])
        ends = plsc.cumsum(lens) + carry
        carry = carry + jnp.sum(lens)
        starts = ends - lens
        for o in range(V - 1):
            plsc.store_scatter(obuf, [starts + (rb + o)], syms,
                               mask=lens > o)


def _fill_body(x_hbm, perms_hbm, syms_hbm, out_hbm, z1_hbm, z2_hbm,
               xin0, xin1, obuf0, obuf1, perms_v, syms_v, vals_row, zbuf,
               isem0, isem1, osem0, osem1, zsem):
    wid = lax.axis_index("c") * 16 + lax.axis_index("s")
    base = wid * RPT
    xins = (xin0, xin1)
    obufs = (obuf0, obuf1)
    isems = (isem0, isem1)
    osems = (osem0, osem1)

    pltpu.sync_copy(perms_hbm.at[pl.ds(base * A, RPT * A)], perms_v)
    pltpu.sync_copy(syms_hbm, syms_v)

    zero16f = jnp.zeros((16,), jnp.float32)
    for i in range(L // 16):
        zbuf[pl.ds(i * 16, 16)] = zero16f

    def start_zeros(k, zsem):
        for i in range(CH):
            row = base + k * CH + i
            pltpu.async_copy(zbuf, z1_hbm.at[row], zsem)
            pltpu.async_copy(zbuf, z2_hbm.at[row], zsem)

    def wait_zeros(k, zsem):
        for i in range(CH):
            row = base + k * CH + i
            pltpu.make_async_copy(zbuf, z1_hbm.at[row], zsem).wait()
            pltpu.make_async_copy(zbuf, z2_hbm.at[row], zsem).wait()

    def start_in(k, xin, isem):
        for i in range(CH):
            pltpu.async_copy(x_hbm.at[base + k * CH + i],
                             xin.at[pl.ds(i * L, L)], isem)

    def wait_in(k, xin, isem):
        for i in range(CH):
            pltpu.make_async_copy(x_hbm.at[base + k * CH + i],
                                  xin.at[pl.ds(i * L, L)], isem).wait()

    def start_out(k, obuf, osem):
        for i in range(CH):
            pltpu.async_copy(obuf.at[pl.ds(i * L, L)],
                             out_hbm.at[base + k * CH + i], osem)

    def wait_out(k, obuf, osem):
        for i in range(CH):
            pltpu.make_async_copy(obuf.at[pl.ds(i * L, L)],
                                  out_hbm.at[base + k * CH + i], osem).wait()

    start_in(0, xin0, isem0)

    @pl.loop(0, NCH // 2)
    def _pair(c):
        for b in range(2):
            k = c * 2 + b
            @pl.when(k < NCH - 1)
            def _():
                start_in(k + 1, xins[1 - b], isems[1 - b])
            @pl.when(k >= 2)
            def _():
                wait_out(k, obufs[b], osems[b])
                wait_zeros(k, zsem)
            start_zeros(k, zsem)
            wait_in(k, xins[b], isems[b])
            xin_b, obuf_b = xins[b], obufs[b]

            @pl.loop(0, CH)
            def _row(r):
                lr = k * CH + r
                _row_fill(xin_b, perms_v, syms_v, vals_row, obuf_b, lr, r)

            start_out(k, obufs[b], osems[b])

    wait_out(NCH - 2, obuf0, osem0)
    wait_out(NCH - 1, obuf1, osem1)
    wait_zeros(NCH - 2, zsem)
    wait_zeros(NCH - 1, zsem)


def _sc_bos(x, perms_flat, symbols):
    kern = functools.partial(
        pl.kernel,
        out_type=[
            jax.ShapeDtypeStruct((B, L), jnp.int32),
            jax.ShapeDtypeStruct((B, L), jnp.float32),
            jax.ShapeDtypeStruct((B, L), jnp.float32),
        ],
        mesh=plsc.VectorSubcoreMesh(core_axis_name="c", subcore_axis_name="s"),
        scratch_types=[
            pltpu.VMEM((CH * L,), jnp.float32),
            pltpu.VMEM((CH * L,), jnp.float32),
            pltpu.VMEM((CH * L,), jnp.int32),
            pltpu.VMEM((CH * L,), jnp.int32),
            pltpu.VMEM((RPT * A,), jnp.int32),
            pltpu.VMEM((A,), jnp.int32),
            pltpu.VMEM((A,), jnp.int32),
            pltpu.VMEM((L,), jnp.float32),
            pltpu.SemaphoreType.DMA,
            pltpu.SemaphoreType.DMA,
            pltpu.SemaphoreType.DMA,
            pltpu.SemaphoreType.DMA,
            pltpu.SemaphoreType.DMA,
        ],
        compiler_params=pltpu.CompilerParams(needs_layout_passes=False),
    )(_fill_body)
    return kern(x, perms_flat, symbols)


def kernel(x, symbols):
    perms_flat = _perms_flat_operand()
    result, z1, z2 = _sc_bos(x, perms_flat, symbols)
    return (result, z1, z2)
